# baseline (speedup 1.0000x reference)
import functools
import math

import jax
import jax.numpy as jnp
from jax.experimental import pallas as pl
from jax.experimental.pallas import tpu as pltpu

_NUM_BUCKETS = 32
_MAX_DISTANCE = 128
_SCALE = 0.125
_LANES = 128


def _stair_kernel(seq, heads, nb, rb, width, embt_ref, s_ref):
    shape = (nb, heads, 8, width)
    t = jax.lax.broadcasted_iota(jnp.int32, shape, 0)
    sr = jax.lax.broadcasted_iota(jnp.int32, shape, 2)
    x = jax.lax.broadcasted_iota(jnp.int32, shape, 3)
    rel = x - sr + (nb - 1 - t) * rb - (seq - 1)
    n = -rel
    num_buckets = _NUM_BUCKETS // 2
    side = jnp.where(n < 0, num_buckets, 0)
    n = jnp.abs(n)
    max_exact = num_buckets // 2
    n_safe = jnp.maximum(n, 1).astype(jnp.float32)
    val_if_large = max_exact + (
        jnp.log(n_safe / max_exact)
        / math.log(_MAX_DISTANCE / max_exact)
        * (num_buckets - max_exact)
    ).astype(jnp.int32)
    val_if_large = jnp.minimum(val_if_large, num_buckets - 1)
    bucket = side + jnp.where(n < max_exact, n, val_if_large)
    acc = jnp.zeros(shape, jnp.float32)
    for b in range(_NUM_BUCKETS):
        v = embt_ref[:, b][None, :, None, None]
        acc = acc + jnp.where(bucket == b, v, 0.0)
    s_ref[...] = acc * _SCALE


def _add_kernel(seq, rb, qk_ref, s_ref, out_ref):
    for rg in range(rb // 8):
        off = (rb - 1) - 8 * rg
        bias = s_ref[0, 0, :, off : off + seq]
        out_ref[0, 0, 8 * rg : 8 * rg + 8, :] = (
            qk_ref[0, 0, 8 * rg : 8 * rg + 8, :] + bias
        )


def kernel(qk_dots, rel_emb):
    batch, heads, seq_i, seq = qk_dots.shape
    assert seq_i == seq and seq % _LANES == 0
    rb = min(seq, 512)
    nb = seq // rb
    width = rb + seq

    embt = jnp.transpose(rel_emb.astype(jnp.float32))
    stair = pl.pallas_call(
        functools.partial(_stair_kernel, seq, heads, nb, rb, width),
        out_shape=jax.ShapeDtypeStruct((nb, heads, 8, width), jnp.float32),
    )(embt)

    return pl.pallas_call(
        functools.partial(_add_kernel, seq, rb),
        grid=(batch, heads, nb),
        in_specs=[
            pl.BlockSpec((1, 1, rb, seq), lambda b, h, t: (b, h, t, 0)),
            pl.BlockSpec((1, 1, 8, width), lambda b, h, t: (t, h, 0, 0)),
        ],
        out_specs=pl.BlockSpec((1, 1, rb, seq), lambda b, h, t: (b, h, t, 0)),
        out_shape=jax.ShapeDtypeStruct((batch, heads, seq, seq), jnp.float32),
        compiler_params=pltpu.CompilerParams(
            dimension_semantics=("parallel", "parallel", "arbitrary")
        ),
    )(qk_dots, stair)
